# Initial kernel scaffold; baseline (speedup 1.0000x reference)
import jax
import jax.numpy as jnp
from jax.experimental import pallas as pl


def kernel(x, router_w, W1, b1, W2, b2, train_scores):
    raise NotImplementedError("write your pallas kernel here")


# speedup vs baseline: 3.2140x; 3.2140x over previous
import functools

import jax
import jax.numpy as jnp
from jax.experimental import pallas as pl
from jax.experimental.pallas import tpu as pltpu

INPUT_DIM = 1024
NUM_EXPERTS = 8
EXPERT_DIM = 1024
TOKEN_TILE = 512
NEG_INF = float("-inf")


def _dense_body(x_ref, rw_ref, ts_ref, w1_ref, b1_ref, w2_ref, b2_ref,
                out_ref, scores_ref, topk_ref, gate_ref, loss_ref,
                fsum, psum):
    t = pl.program_id(0)
    e = pl.program_id(1)
    nt = pl.num_programs(0)

    @pl.when(e == 0)
    def _router():
        x = x_ref[...]
        s = jax.lax.dot_general(x, rw_ref[...],
                                (((1,), (1,)), ((), ())),
                                preferred_element_type=jnp.float32)
        scores_ref[...] = s
        iota8 = jax.lax.broadcasted_iota(jnp.int32, (TOKEN_TILE, NUM_EXPERTS), 1)
        m0 = jnp.max(s, axis=1, keepdims=True)
        e0 = jnp.min(jnp.where(s == m0, iota8, NUM_EXPERTS), axis=1, keepdims=True)
        s_wo = jnp.where(iota8 == e0, NEG_INF, s)
        m1 = jnp.max(s_wo, axis=1, keepdims=True)
        e1 = jnp.min(jnp.where((s == m1) & (iota8 != e0), iota8, NUM_EXPERTS),
                     axis=1, keepdims=True)
        topk_ref[...] = jnp.concatenate([e0, e1], axis=1)
        mask = ((iota8 == e0) | (iota8 == e1)).astype(jnp.float32)
        ts = ts_ref[...]
        tw = jnp.exp(ts - jnp.max(ts, axis=1, keepdims=True))
        tw = tw / jnp.sum(tw, axis=1, keepdims=True)
        gate_un = mask * tw
        den = jnp.maximum(jnp.sum(gate_un, axis=1, keepdims=True), 1e-8)
        gate_ref[...] = gate_un / den
        p = jnp.exp(s - m0)
        p = p / jnp.sum(p, axis=1, keepdims=True)

        @pl.when(t == 0)
        def _init():
            fsum[...] = jnp.zeros_like(fsum)
            psum[...] = jnp.zeros_like(psum)

        fsum[...] += jnp.sum(mask, axis=0, keepdims=True)
        psum[...] += jnp.sum(p, axis=0, keepdims=True)

        @pl.when(t == nt - 1)
        def _loss():
            n_tok = nt * TOKEN_TILE
            loss_ref[...] = (NUM_EXPERTS / (n_tok * n_tok)) * jnp.sum(
                fsum[...] * psum[...], keepdims=True)

    x = x_ref[...]
    h = jax.lax.dot_general(x, w1_ref[0], (((1,), (1,)), ((), ())),
                            preferred_element_type=jnp.float32)
    h = h + b1_ref[0]
    a = 0.5 * h * (1.0 + jax.lax.erf(h * 0.7071067811865476))
    eo = jax.lax.dot_general(a, w2_ref[0], (((1,), (1,)), ((), ())),
                             preferred_element_type=jnp.float32)
    eo = eo + b2_ref[0]
    iota8 = jax.lax.broadcasted_iota(jnp.int32, (TOKEN_TILE, NUM_EXPERTS), 1)
    ge = jnp.sum(gate_ref[...] * (iota8 == e).astype(jnp.float32),
                 axis=1, keepdims=True)

    @pl.when(e == 0)
    def _first():
        out_ref[...] = ge * eo

    @pl.when(e > 0)
    def _acc():
        out_ref[...] += ge * eo


@functools.partial(jax.jit, static_argnames=("interpret",))
def _moe_dense(xf, router_w, W1, b1, W2, b2, ts2d, interpret=False):
    n = xf.shape[0]
    nt = n // TOKEN_TILE
    grid = (nt, NUM_EXPERTS)
    out_shapes = (
        jax.ShapeDtypeStruct((n, INPUT_DIM), jnp.float32),
        jax.ShapeDtypeStruct((n, NUM_EXPERTS), jnp.float32),
        jax.ShapeDtypeStruct((n, 2), jnp.int32),
        jax.ShapeDtypeStruct((n, NUM_EXPERTS), jnp.float32),
        jax.ShapeDtypeStruct((1, 1), jnp.float32),
    )
    in_specs = [
        pl.BlockSpec((TOKEN_TILE, INPUT_DIM), lambda t, e: (t, 0)),
        pl.BlockSpec((NUM_EXPERTS, INPUT_DIM), lambda t, e: (0, 0)),
        pl.BlockSpec((1, NUM_EXPERTS), lambda t, e: (0, 0)),
        pl.BlockSpec((1, EXPERT_DIM, INPUT_DIM), lambda t, e: (e, 0, 0)),
        pl.BlockSpec((1, 1, EXPERT_DIM), lambda t, e: (e, 0, 0)),
        pl.BlockSpec((1, INPUT_DIM, EXPERT_DIM), lambda t, e: (e, 0, 0)),
        pl.BlockSpec((1, 1, INPUT_DIM), lambda t, e: (e, 0, 0)),
    ]
    out_specs = (
        pl.BlockSpec((TOKEN_TILE, INPUT_DIM), lambda t, e: (t, 0)),
        pl.BlockSpec((TOKEN_TILE, NUM_EXPERTS), lambda t, e: (t, 0)),
        pl.BlockSpec((TOKEN_TILE, 2), lambda t, e: (t, 0)),
        pl.BlockSpec((TOKEN_TILE, NUM_EXPERTS), lambda t, e: (t, 0)),
        pl.BlockSpec((1, 1), lambda t, e: (0, 0)),
    )
    return pl.pallas_call(
        _dense_body,
        grid=grid,
        in_specs=in_specs,
        out_specs=out_specs,
        out_shape=out_shapes,
        scratch_shapes=[
            pltpu.VMEM((1, NUM_EXPERTS), jnp.float32),
            pltpu.VMEM((1, NUM_EXPERTS), jnp.float32),
        ],
        compiler_params=pltpu.CompilerParams(
            dimension_semantics=("arbitrary", "arbitrary")),
        interpret=interpret,
    )(xf, router_w, ts2d, W1, b1.reshape(NUM_EXPERTS, 1, EXPERT_DIM),
      W2, b2.reshape(NUM_EXPERTS, 1, INPUT_DIM))


def kernel(x, router_w, W1, b1, W2, b2, train_scores):
    orig_shape = x.shape
    xf = x.reshape(-1, orig_shape[-1])
    ts2d = train_scores.reshape(1, NUM_EXPERTS)
    out, scores, topk, gate, loss = _moe_dense(xf, router_w, W1, b1, W2, b2, ts2d)
    return (out.reshape(orig_shape),
            loss[0, 0],
            scores.reshape(orig_shape[:-1] + (NUM_EXPERTS,)),
            topk.reshape(orig_shape[:-1] + (2,)),
            gate.reshape(orig_shape[:-1] + (NUM_EXPERTS,)),
            train_scores)
